# Initial kernel scaffold; baseline (speedup 1.0000x reference)
import jax
import jax.numpy as jnp
from jax.experimental import pallas as pl


def kernel(data, user_sim, service_sim, user_emb, service_emb, cnn_w, cnn_b, scnn_w, scnn_b, wi, bi, wf, bf, fc_w, fc_b):
    raise NotImplementedError("write your pallas kernel here")


# speedup vs baseline: 5.8382x; 5.8382x over previous
import functools

import jax
import jax.numpy as jnp
from jax.experimental import pallas as pl
from jax.experimental.pallas import tpu as pltpu

K = 32
TAPS = 7
R1 = 8
UB = 16
BS = 128

_NEG = float("-inf")


def _topk_kernel(x_ref, out_ref):
    x = x_ref[...].astype(jnp.float32)
    lane_n = jax.lax.broadcasted_iota(jnp.int32, x.shape, 1)
    lane128 = jax.lax.broadcasted_iota(jnp.int32, (R1, 128), 1)

    lane_f = lane_n.astype(jnp.float32)
    idx_acc = jnp.zeros((R1, 128), jnp.int32)
    for k in range(K):
        m = jnp.max(x, axis=1, keepdims=True)
        a_f = jnp.min(jnp.where(x == m, lane_f, jnp.float32(1e9)),
                      axis=1, keepdims=True)
        a = a_f.astype(jnp.int32)
        x = jnp.where(lane_n == a, _NEG, x)
        idx_acc = idx_acc + jnp.where(lane128 == k, a, 0)

    iv = jnp.where(lane128 < K, idx_acc, 0)
    i3 = iv + pltpu.roll(iv, 96, axis=1)
    rank = jnp.zeros((R1, 128), jnp.int32)
    for j in range(1, K):
        pj = pltpu.roll(i3, j, axis=1)
        rank = rank + jnp.where(pj < idx_acc, 1, 0)

    srt = jnp.zeros((R1, 128), jnp.int32)
    for j in range(K):
        srt = srt + jnp.where(lane128 == rank[:, j:j + 1], idx_acc[:, j:j + 1], 0)
    out_ref[...] = srt[:, :K]


def _topk_call(sim):
    n = sim.shape[0]
    return pl.pallas_call(
        _topk_kernel,
        grid=(n // R1,),
        in_specs=[pl.BlockSpec((R1, n), lambda i: (i, 0))],
        out_specs=pl.BlockSpec((R1, K), lambda i: (i, 0)),
        out_shape=jax.ShapeDtypeStruct((n, K), jnp.int32),
        compiler_params=pltpu.CompilerParams(dimension_semantics=("parallel",)),
    )(sim)


def _conv_kernel(idx_ref, emb7_ref, w_ref, b_ref, out_ref, p_ref):
    h = out_ref.shape[1]
    for u in range(UB):
        for j in range(K):
            r = idx_ref[u, j]
            p_ref[pl.ds(j * TAPS, TAPS), pl.ds(u * h, h)] = emb7_ref[r]
    conv = jax.lax.dot_general(w_ref[...], p_ref[...], (((1,), (0,)), ((), ())),
                               preferred_element_type=jnp.float32)
    for u in range(UB):
        out_ref[u, :] = conv[u, u * h:(u + 1) * h] + b_ref[0, 0]


def _conv_call(nbr, emb7, wrep, bias):
    n, _, h = emb7.shape
    return pl.pallas_call(
        _conv_kernel,
        grid=(n // UB,),
        in_specs=[
            pl.BlockSpec((UB, K), lambda i: (i, 0), memory_space=pltpu.SMEM),
            pl.BlockSpec((n, TAPS, h), lambda i: (0, 0, 0)),
            pl.BlockSpec((UB, K * TAPS), lambda i: (0, 0)),
            pl.BlockSpec((1, 1), lambda i: (0, 0), memory_space=pltpu.SMEM),
        ],
        out_specs=pl.BlockSpec((UB, h), lambda i: (i, 0)),
        out_shape=jax.ShapeDtypeStruct((n, h), jnp.float32),
        scratch_shapes=[pltpu.VMEM((K * TAPS, UB * h), jnp.float32)],
        compiler_params=pltpu.CompilerParams(dimension_semantics=("parallel",)),
    )(nbr, emb7, wrep, bias)


def _assemble_kernel(ids_ref, ue_ref, cu_ref, se_ref, ds_ref,
                     wft_ref, bf_ref, wit_ref, bi_ref, fcwt_ref, fcb_ref,
                     out_ref, a_ref, b_ref, c_ref, d_ref):
    h = ue_ref.shape[2]
    for s in range(BS):
        u = ids_ref[s, 0]
        v = ids_ref[s, 1]
        a_ref[s, :] = ue_ref[u, 0, :]
        c_ref[s, :] = cu_ref[u, 0, :]
        b_ref[s, :] = se_ref[v, 0, :]
        d_ref[s, :] = ds_ref[v, 0, :]

    a, b, c, d = a_ref[...], b_ref[...], c_ref[...], d_ref[...]
    wft, wit = wft_ref[...], wit_ref[...]
    bfv, biv = bf_ref[...], bi_ref[...]

    tempts = []
    n2cols = []
    for (x, y) in ((a, b), (a, c), (a, d), (b, c), (b, d), (c, d)):
        sp = x + y
        f = jax.nn.sigmoid(
            jax.lax.dot(sp, wft, preferred_element_type=jnp.float32) + bfv)
        tt = jax.lax.dot(sp, wit, preferred_element_type=jnp.float32) + biv
        te = jnp.maximum(f * tt + (1.0 - f) * sp, 0.0)
        tempts.append(te)
        n2cols.append(jnp.sum(te * te, axis=1, keepdims=True))
    n2 = jnp.concatenate(n2cols, axis=1)

    lane6 = jax.lax.broadcasted_iota(jnp.int32, (BS, 6), 1)
    out = jnp.broadcast_to(fcb_ref[...], (BS, fcb_ref.shape[1]))
    for r in range(4):
        m = jnp.max(n2, axis=1, keepdims=True)
        cand = jnp.where(n2 == m, lane6, 6)
        pidx = jnp.min(cand, axis=1, keepdims=True)
        sel = jnp.zeros_like(a)
        for p in range(6):
            sel = sel + jnp.where(pidx == p, 1.0, 0.0) * tempts[p]
        out = out + jax.lax.dot(sel, fcwt_ref[pl.ds(r * h, h), :],
                                preferred_element_type=jnp.float32)
        n2 = jnp.where(lane6 == pidx, _NEG, n2)
    out_ref[...] = out


def _assemble_call(ids, ue3, cu3, se3, ds3, wft, bfv, wit, biv, fcwt, fcbv):
    bsz = ids.shape[0]
    n, _, h = ue3.shape
    out_dim = fcwt.shape[1]
    full = lambda shape: pl.BlockSpec(shape, lambda i: tuple(0 for _ in shape))
    return pl.pallas_call(
        _assemble_kernel,
        grid=(bsz // BS,),
        in_specs=[
            pl.BlockSpec((BS, 2), lambda i: (i, 0), memory_space=pltpu.SMEM),
            full((n, 1, h)), full((n, 1, h)), full((n, 1, h)), full((n, 1, h)),
            full((h, h)), full((1, h)), full((h, h)), full((1, h)),
            full((4 * h, out_dim)), full((1, out_dim)),
        ],
        out_specs=pl.BlockSpec((BS, out_dim), lambda i: (i, 0)),
        out_shape=jax.ShapeDtypeStruct((bsz, out_dim), jnp.float32),
        scratch_shapes=[pltpu.VMEM((BS, h), jnp.float32) for _ in range(4)],
        compiler_params=pltpu.CompilerParams(dimension_semantics=("parallel",)),
    )(ids, ue3, cu3, se3, ds3, wft, bfv, wit, biv, fcwt, fcbv)


def kernel(data, user_sim, service_sim, user_emb, service_emb,
           cnn_w, cnn_b, scnn_w, scnn_b, wi, bi, wf, bf, fc_w, fc_b):
    n, h = user_emb.shape
    uid = data[:, 1].astype(jnp.int32)
    sid = data[:, 2].astype(jnp.int32)
    ids = jnp.stack([uid, sid], axis=1)

    u_map = _topk_call(user_sim)
    s_map = _topk_call(service_sim)

    ue3 = user_emb.reshape(n, 1, h)
    se3 = service_emb.reshape(n, 1, h)
    uep = jnp.pad(user_emb, ((0, 0), (3, 3)))
    sep = jnp.pad(service_emb, ((0, 0), (3, 3)))
    ue7 = jnp.stack([uep[:, t:t + h] for t in range(TAPS)], axis=1)
    se7 = jnp.stack([sep[:, t:t + h] for t in range(TAPS)], axis=1)
    wrep_u = jnp.broadcast_to(cnn_w[0, 0].reshape(1, K * TAPS), (UB, K * TAPS))
    wrep_s = jnp.broadcast_to(scnn_w[0, 0].reshape(1, K * TAPS), (UB, K * TAPS))
    cu = _conv_call(u_map, ue7, wrep_u, cnn_b.reshape(1, 1))
    ds = _conv_call(s_map, se7, wrep_s, scnn_b.reshape(1, 1))

    out = _assemble_call(
        ids, ue3, cu.reshape(n, 1, h), se3, ds.reshape(n, 1, h),
        wf.T, bf.reshape(1, h), wi.T, bi.reshape(1, h),
        fc_w.T, fc_b.reshape(1, -1))
    return out
